# baseline (speedup 1.0000x reference)
import dataclasses
import functools

import jax
import jax.numpy as jnp
from jax.experimental import pallas as pl
from jax.experimental.pallas import tpu as pltpu
from jax.experimental.pallas import tpu_sc as plsc

_T = 2048
_D = 1024
_E = 8
_F = 2048
_K = 2
_C = (_T * _K) // _E
_EC = _E * _C

_NUM_WORKERS = 32
_ROWS_PER_CHUNK = 32
_DP = _D // 2


def _rne_bf16_bits(a):
    u = jax.lax.bitcast_convert_type(a, jnp.int32)
    rounded = u + 0x7FFF + (jax.lax.shift_right_logical(u, 16) & 1)
    return rounded


def _pack_bf16(a):
    n2 = a.shape[1] // 2
    lo = jax.lax.shift_right_logical(_rne_bf16_bits(a[:, :n2]), 16)
    hi = _rne_bf16_bits(a[:, n2:]) & jnp.int32(-65536)
    return jax.lax.bitcast_convert_type(lo | hi, jnp.float32)


def _unpack_bf16(p):
    wi = jax.lax.bitcast_convert_type(p, jnp.int32)
    lo = jax.lax.bitcast_convert_type(wi << 16, jnp.float32)
    hi = jax.lax.bitcast_convert_type(wi & jnp.int32(-65536), jnp.float32)
    return jnp.concatenate([lo, hi], axis=1)


def _cumsum_lanes(a):
    n = a.shape[1]
    k = 1
    while k < n:
        shifted = jnp.concatenate(
            [jnp.zeros((a.shape[0], k), a.dtype), a[:, :-k]], axis=1)
        a = a + shifted
        k *= 2
    return a


def _gate_body(x_ref, wg_ref, fi_ref, kf_ref, wc_ref, xb_ref):
    x = x_ref[...]
    xb_ref[...] = _pack_bf16(x)
    wg = wg_ref[...]
    logits = jnp.dot(x, wg, preferred_element_type=jnp.float32)
    lt = logits.T
    mx = jnp.max(lt, axis=0, keepdims=True)
    eg = jnp.exp(lt - mx)
    gates = eg / jnp.sum(eg, axis=0, keepdims=True)

    iota = jax.lax.broadcasted_iota(jnp.int32, (_E, _T), 0)
    m1 = jnp.max(gates, axis=0, keepdims=True)
    idx1 = jnp.min(jnp.where(gates == m1, iota, _E), axis=0, keepdims=True)
    g2 = jnp.where(iota == idx1, -jnp.inf, gates)
    m2 = jnp.max(g2, axis=0, keepdims=True)
    idx2 = jnp.min(jnp.where(g2 == m2, iota, _E), axis=0, keepdims=True)

    s = m1 + m2 + 1e-9
    w1v = m1 / s
    w2v = m2 / s

    oh1 = (iota == idx1).astype(jnp.int32)
    oh2 = (iota == idx2).astype(jnp.int32)
    inc1 = _cumsum_lanes(oh1)
    inc2 = _cumsum_lanes(oh2)
    cnt1 = inc1[:, _T - 1:_T]
    pos1 = jnp.sum(jnp.where(oh1 == 1, inc1 - 1, 0), axis=0, keepdims=True)
    pos2 = jnp.sum(jnp.where(oh2 == 1, inc2 - 1 + cnt1, 0), axis=0,
                   keepdims=True)

    keep1 = pos1 < _C
    keep2 = pos2 < _C
    fi1 = jnp.where(keep1, idx1 * _C + pos1, _EC - 1)
    fi2 = jnp.where(keep2, idx2 * _C + pos2, _EC - 1)

    fi_ref[pl.ds(0, _T)] = fi1.reshape(_T)
    fi_ref[pl.ds(_T, _T)] = fi2.reshape(_T)
    kf_ref[pl.ds(0, _T)] = keep1.astype(jnp.int32).reshape(_T)
    kf_ref[pl.ds(_T, _T)] = keep2.astype(jnp.int32).reshape(_T)
    wcT = jnp.concatenate([jnp.where(keep1, w1v, 0.0),
                           jnp.where(keep2, w2v, 0.0)], axis=0)
    wc_ref[...] = wcT.T


def _gate(x, wg, interpret=False):
    return pl.pallas_call(
        _gate_body,
        out_shape=(jax.ShapeDtypeStruct((_K * _T,), jnp.int32),
                   jax.ShapeDtypeStruct((_K * _T,), jnp.int32),
                   jax.ShapeDtypeStruct((_T, _K), jnp.float32),
                   jax.ShapeDtypeStruct((_T, _DP), jnp.float32)),
        interpret=interpret,
    )(x, wg)


def _sc_compiler_params():
    cp = pltpu.CompilerParams()
    if "needs_layout_passes" in pltpu.CompilerParams.__dataclass_fields__:
        cp = dataclasses.replace(cp, needs_layout_passes=False)
    return cp


def _mesh():
    return plsc.VectorSubcoreMesh(core_axis_name="c", subcore_axis_name="s")


def _make_dispatch_kernel():
    per_worker = _EC // _NUM_WORKERS
    ch = _ROWS_PER_CHUNK
    n_chunks = per_worker // ch

    @functools.partial(
        pl.kernel,
        out_type=jax.ShapeDtypeStruct((_EC, _DP), jnp.float32),
        mesh=_mesh(),
        compiler_params=_sc_compiler_params(),
        scratch_types=[pltpu.VMEM((_EC,), jnp.int32),
                       pltpu.VMEM((_EC,), jnp.int32),
                       pltpu.VMEM((_EC,), jnp.int32),
                       pltpu.VMEM((ch, _DP), jnp.float32),
                       pltpu.VMEM((ch, _DP), jnp.float32),
                       pltpu.SemaphoreType.DMA,
                       pltpu.SemaphoreType.DMA,
                       pltpu.SemaphoreType.DMA,
                       pltpu.SemaphoreType.DMA],
    )
    def dispatch_kernel(x_hbm, fi_hbm, kf_hbm, buf_hbm,
                        fi_v, kf_v, inv_v, b0, b1, gs0, gs1, ws0, ws1):
        wid = jax.lax.axis_index("s") * 2 + jax.lax.axis_index("c")
        base = wid * per_worker
        pltpu.sync_copy(fi_hbm, fi_v)
        pltpu.sync_copy(kf_hbm, kf_v)

        @pl.loop(0, per_worker // 16)
        def _(i):
            inv_v[pl.ds(base + i * 16, 16)] = jnp.zeros((16,), jnp.int32)

        for s in (0, 1):
            @pl.loop(0, _T // 16)
            def _(i, s=s):
                eb = i * 16
                f = fi_v[pl.ds(s * _T + eb, 16)]
                kf = kf_v[pl.ds(s * _T + eb, 16)]
                tok = jax.lax.iota(jnp.int32, 16) + eb
                plsc.store_scatter(inv_v, [f], tok, mask=kf == 1)

        my_idx = inv_v.at[pl.ds(base, per_worker)]
        bufs = (b0, b1)
        gsems = (gs0, gs1)
        wsems = (ws0, ws1)
        writes = [None, None]
        gathers = [None, None]
        for c in range(n_chunks):
            s = c % 2
            if writes[s] is not None:
                writes[s].wait()
            gathers[s] = pltpu.async_copy(
                x_hbm.at[my_idx.at[pl.ds(c * ch, ch)]], bufs[s], gsems[s])
            if c % 2 == 1:
                for s2 in (0, 1):
                    gathers[s2].wait()
                    writes[s2] = pltpu.async_copy(
                        bufs[s2],
                        buf_hbm.at[pl.ds(base + (c - 1 + s2) * ch, ch)],
                        wsems[s2])
        for s2 in (0, 1):
            if writes[s2] is not None:
                writes[s2].wait()

    return dispatch_kernel


def _dispatch(x, fi_flat, kf_flat):
    return _make_dispatch_kernel()(x, fi_flat, kf_flat)


def _make_combine_gather_kernel():
    per_worker = _EC // _NUM_WORKERS
    ch = _ROWS_PER_CHUNK
    n_chunks = per_worker // ch

    @functools.partial(
        pl.kernel,
        out_type=jax.ShapeDtypeStruct((_EC, _DP), jnp.float32),
        mesh=_mesh(),
        compiler_params=_sc_compiler_params(),
        scratch_types=[pltpu.VMEM((per_worker,), jnp.int32),
                       pltpu.VMEM((ch, _DP), jnp.float32),
                       pltpu.VMEM((ch, _DP), jnp.float32),
                       pltpu.SemaphoreType.DMA,
                       pltpu.SemaphoreType.DMA,
                       pltpu.SemaphoreType.DMA,
                       pltpu.SemaphoreType.DMA],
    )
    def combine_kernel(eo_hbm, fi_hbm, out_hbm,
                       idx_v, b0, b1, gs0, gs1, ws0, ws1):
        wid = jax.lax.axis_index("s") * 2 + jax.lax.axis_index("c")
        base = wid * per_worker

        pltpu.sync_copy(fi_hbm.at[pl.ds(base, per_worker)], idx_v)
        bufs = (b0, b1)
        gsems = (gs0, gs1)
        wsems = (ws0, ws1)
        writes = [None, None]
        gathers = [None, None]
        for c in range(n_chunks):
            s = c % 2
            if writes[s] is not None:
                writes[s].wait()
            gathers[s] = pltpu.async_copy(
                eo_hbm.at[idx_v.at[pl.ds(c * ch, ch)]], bufs[s], gsems[s])
            if c % 2 == 1:
                for s2 in (0, 1):
                    gathers[s2].wait()
                    writes[s2] = pltpu.async_copy(
                        bufs[s2],
                        out_hbm.at[pl.ds(base + (c - 1 + s2) * ch, ch)],
                        wsems[s2])
        for s2 in (0, 1):
            if writes[s2] is not None:
                writes[s2].wait()

    return combine_kernel


def _gather_rows(table, idx):
    return _make_combine_gather_kernel()(table, idx)


_NJ = 1


def _ffn_body(xin_ref, w1_ref, b1_ref, w2_ref, b2_ref, out_ref):
    xin = _unpack_bf16(xin_ref[...])
    h = jnp.maximum(
        jnp.dot(xin, w1_ref[0], preferred_element_type=jnp.float32)
        + b1_ref[0], 0.0)
    part = jnp.dot(h, w2_ref[0], preferred_element_type=jnp.float32)
    out_ref[...] = _pack_bf16(part + b2_ref[0])


def _ffn(buf, w1, b1r, w2, b2r, interpret=False):
    return pl.pallas_call(
        _ffn_body,
        grid=(_E,),
        in_specs=[
            pl.BlockSpec((_C, _DP), lambda e: (e, 0)),
            pl.BlockSpec((1, _D, _F), lambda e: (e, 0, 0)),
            pl.BlockSpec((1, 1, _F), lambda e: (e, 0, 0)),
            pl.BlockSpec((1, _F, _D), lambda e: (e, 0, 0)),
            pl.BlockSpec((1, 1, _D), lambda e: (e, 0, 0)),
        ],
        out_specs=pl.BlockSpec((_C, _DP), lambda e: (e, 0)),
        out_shape=jax.ShapeDtypeStruct((_EC, _DP), jnp.float32),
        interpret=interpret,
    )(buf, w1, b1r, w2, b2r)


_TBLK = 256


def _combine_body(g0_ref, g1_ref, wc_ref, y_ref):
    w0 = wc_ref[:, 0:1]
    w1 = wc_ref[:, 1:2]
    g0 = _unpack_bf16(g0_ref[...])
    g1 = _unpack_bf16(g1_ref[...])
    y_ref[...] = (jnp.where(w0 > 0, w0 * g0, 0.0)
                  + jnp.where(w1 > 0, w1 * g1, 0.0))


def _combine(g, wc, interpret=False):
    nb = _T // _TBLK
    return pl.pallas_call(
        _combine_body,
        grid=(nb,),
        in_specs=[
            pl.BlockSpec((_TBLK, _DP), lambda t: (t, 0)),
            pl.BlockSpec((_TBLK, _DP), lambda t: (t + nb, 0)),
            pl.BlockSpec((_TBLK, _K), lambda t: (t, 0)),
        ],
        out_specs=pl.BlockSpec((_TBLK, _D), lambda t: (t, 0)),
        out_shape=jax.ShapeDtypeStruct((_T, _D), jnp.float32),
        interpret=interpret,
    )(g, g, wc)


def kernel(x, wg, w1, b1, w2, b2):
    fi, kf, wc, xb = _gate(x, wg)
    buf = _dispatch(xb, fi, kf)
    eo = _ffn(buf, w1, b1.reshape(_E, 1, _F), w2, b2.reshape(_E, 1, _D))
    g = _gather_rows(eo, fi)
    return _combine(g, wc)


# speedup vs baseline: 1.0344x; 1.0344x over previous
import dataclasses
import functools

import jax
import jax.numpy as jnp
from jax.experimental import pallas as pl
from jax.experimental.pallas import tpu as pltpu
from jax.experimental.pallas import tpu_sc as plsc

_T = 2048
_D = 1024
_E = 8
_F = 2048
_K = 2
_C = (_T * _K) // _E
_EC = _E * _C

_NUM_WORKERS = 32
_ROWS_PER_CHUNK = 32
_DP = _D // 2


def _pack_bf16(a):
    n2 = a.shape[1] // 2
    b = a.astype(jnp.bfloat16)
    lo = (jax.lax.bitcast_convert_type(b[:, :n2], jnp.int16)
          .astype(jnp.int32) & 0xFFFF)
    hi = jax.lax.bitcast_convert_type(b[:, n2:], jnp.int16).astype(jnp.int32)
    return jax.lax.bitcast_convert_type(lo | (hi << 16), jnp.float32)


def _unpack_bf16(p):
    wi = jax.lax.bitcast_convert_type(p, jnp.int32)
    lo = jax.lax.bitcast_convert_type(
        (wi & 0xFFFF).astype(jnp.int16), jnp.bfloat16).astype(jnp.float32)
    hi = jax.lax.bitcast_convert_type(
        jax.lax.shift_right_logical(wi, 16).astype(jnp.int16),
        jnp.bfloat16).astype(jnp.float32)
    return jnp.concatenate([lo, hi], axis=1)


def _cumsum_lanes(a):
    n = a.shape[1]
    k = 1
    while k < n:
        shifted = jnp.concatenate(
            [jnp.zeros((a.shape[0], k), a.dtype), a[:, :-k]], axis=1)
        a = a + shifted
        k *= 2
    return a


def _gate_body(x_ref, wg_ref, fi_ref, kf_ref, wc_ref, xb_ref):
    x = x_ref[...]
    xb_ref[...] = _pack_bf16(x)
    wg = wg_ref[...]
    logits = jnp.dot(x, wg, preferred_element_type=jnp.float32)
    lt = logits.T
    mx = jnp.max(lt, axis=0, keepdims=True)
    eg = jnp.exp(lt - mx)
    gates = eg / jnp.sum(eg, axis=0, keepdims=True)

    iota = jax.lax.broadcasted_iota(jnp.int32, (_E, _T), 0)
    m1 = jnp.max(gates, axis=0, keepdims=True)
    idx1 = jnp.min(jnp.where(gates == m1, iota, _E), axis=0, keepdims=True)
    g2 = jnp.where(iota == idx1, -jnp.inf, gates)
    m2 = jnp.max(g2, axis=0, keepdims=True)
    idx2 = jnp.min(jnp.where(g2 == m2, iota, _E), axis=0, keepdims=True)

    s = m1 + m2 + 1e-9
    w1v = m1 / s
    w2v = m2 / s

    oh1 = (iota == idx1).astype(jnp.int32)
    oh2 = (iota == idx2).astype(jnp.int32)
    inc1 = _cumsum_lanes(oh1)
    inc2 = _cumsum_lanes(oh2)
    cnt1 = inc1[:, _T - 1:_T]
    pos1 = jnp.sum(jnp.where(oh1 == 1, inc1 - 1, 0), axis=0, keepdims=True)
    pos2 = jnp.sum(jnp.where(oh2 == 1, inc2 - 1 + cnt1, 0), axis=0,
                   keepdims=True)

    keep1 = pos1 < _C
    keep2 = pos2 < _C
    fi1 = jnp.where(keep1, idx1 * _C + pos1, _EC - 1)
    fi2 = jnp.where(keep2, idx2 * _C + pos2, _EC - 1)

    fi_ref[pl.ds(0, _T)] = fi1.reshape(_T)
    fi_ref[pl.ds(_T, _T)] = fi2.reshape(_T)
    kf_ref[pl.ds(0, _T)] = keep1.astype(jnp.int32).reshape(_T)
    kf_ref[pl.ds(_T, _T)] = keep2.astype(jnp.int32).reshape(_T)
    wcT = jnp.concatenate([jnp.where(keep1, w1v, 0.0),
                           jnp.where(keep2, w2v, 0.0)], axis=0)
    wc_ref[...] = wcT.T


def _gate(x, wg, interpret=False):
    return pl.pallas_call(
        _gate_body,
        out_shape=(jax.ShapeDtypeStruct((_K * _T,), jnp.int32),
                   jax.ShapeDtypeStruct((_K * _T,), jnp.int32),
                   jax.ShapeDtypeStruct((_T, _K), jnp.float32),
                   jax.ShapeDtypeStruct((_T, _DP), jnp.float32)),
        interpret=interpret,
    )(x, wg)


def _sc_compiler_params():
    cp = pltpu.CompilerParams()
    if "needs_layout_passes" in pltpu.CompilerParams.__dataclass_fields__:
        cp = dataclasses.replace(cp, needs_layout_passes=False)
    return cp


def _mesh():
    return plsc.VectorSubcoreMesh(core_axis_name="c", subcore_axis_name="s")


def _make_dispatch_kernel():
    per_worker = _EC // _NUM_WORKERS
    ch = _ROWS_PER_CHUNK
    n_chunks = per_worker // ch

    @functools.partial(
        pl.kernel,
        out_type=jax.ShapeDtypeStruct((_EC, _DP), jnp.float32),
        mesh=_mesh(),
        compiler_params=_sc_compiler_params(),
        scratch_types=[pltpu.VMEM((_EC,), jnp.int32),
                       pltpu.VMEM((_EC,), jnp.int32),
                       pltpu.VMEM((_EC,), jnp.int32),
                       pltpu.VMEM((ch, _DP), jnp.float32),
                       pltpu.VMEM((ch, _DP), jnp.float32),
                       pltpu.SemaphoreType.DMA,
                       pltpu.SemaphoreType.DMA,
                       pltpu.SemaphoreType.DMA,
                       pltpu.SemaphoreType.DMA],
    )
    def dispatch_kernel(x_hbm, fi_hbm, kf_hbm, buf_hbm,
                        fi_v, kf_v, inv_v, b0, b1, gs0, gs1, ws0, ws1):
        wid = jax.lax.axis_index("s") * 2 + jax.lax.axis_index("c")
        base = wid * per_worker
        pltpu.sync_copy(fi_hbm, fi_v)
        pltpu.sync_copy(kf_hbm, kf_v)

        @pl.loop(0, per_worker // 16)
        def _(i):
            inv_v[pl.ds(base + i * 16, 16)] = jnp.zeros((16,), jnp.int32)

        for s in (0, 1):
            @pl.loop(0, _T // 16)
            def _(i, s=s):
                eb = i * 16
                f = fi_v[pl.ds(s * _T + eb, 16)]
                kf = kf_v[pl.ds(s * _T + eb, 16)]
                tok = jax.lax.iota(jnp.int32, 16) + eb
                plsc.store_scatter(inv_v, [f], tok, mask=kf == 1)

        my_idx = inv_v.at[pl.ds(base, per_worker)]
        bufs = (b0, b1)
        gsems = (gs0, gs1)
        wsems = (ws0, ws1)
        writes = [None, None]
        gathers = [None, None]
        for c in range(n_chunks):
            s = c % 2
            if writes[s] is not None:
                writes[s].wait()
            gathers[s] = pltpu.async_copy(
                x_hbm.at[my_idx.at[pl.ds(c * ch, ch)]], bufs[s], gsems[s])
            if c % 2 == 1:
                for s2 in (0, 1):
                    gathers[s2].wait()
                    writes[s2] = pltpu.async_copy(
                        bufs[s2],
                        buf_hbm.at[pl.ds(base + (c - 1 + s2) * ch, ch)],
                        wsems[s2])
        for s2 in (0, 1):
            if writes[s2] is not None:
                writes[s2].wait()

    return dispatch_kernel


def _dispatch(x, fi_flat, kf_flat):
    return _make_dispatch_kernel()(x, fi_flat, kf_flat)


def _make_combine_gather_kernel():
    per_worker = _EC // _NUM_WORKERS
    ch = _ROWS_PER_CHUNK
    n_chunks = per_worker // ch

    @functools.partial(
        pl.kernel,
        out_type=jax.ShapeDtypeStruct((_EC, _DP), jnp.float32),
        mesh=_mesh(),
        compiler_params=_sc_compiler_params(),
        scratch_types=[pltpu.VMEM((per_worker,), jnp.int32),
                       pltpu.VMEM((ch, _DP), jnp.float32),
                       pltpu.VMEM((ch, _DP), jnp.float32),
                       pltpu.SemaphoreType.DMA,
                       pltpu.SemaphoreType.DMA,
                       pltpu.SemaphoreType.DMA,
                       pltpu.SemaphoreType.DMA],
    )
    def combine_kernel(eo_hbm, fi_hbm, out_hbm,
                       idx_v, b0, b1, gs0, gs1, ws0, ws1):
        wid = jax.lax.axis_index("s") * 2 + jax.lax.axis_index("c")
        base = wid * per_worker

        pltpu.sync_copy(fi_hbm.at[pl.ds(base, per_worker)], idx_v)
        bufs = (b0, b1)
        gsems = (gs0, gs1)
        wsems = (ws0, ws1)
        writes = [None, None]
        gathers = [None, None]
        for c in range(n_chunks):
            s = c % 2
            if writes[s] is not None:
                writes[s].wait()
            gathers[s] = pltpu.async_copy(
                eo_hbm.at[idx_v.at[pl.ds(c * ch, ch)]], bufs[s], gsems[s])
            if c % 2 == 1:
                for s2 in (0, 1):
                    gathers[s2].wait()
                    writes[s2] = pltpu.async_copy(
                        bufs[s2],
                        out_hbm.at[pl.ds(base + (c - 1 + s2) * ch, ch)],
                        wsems[s2])
        for s2 in (0, 1):
            if writes[s2] is not None:
                writes[s2].wait()

    return combine_kernel


def _gather_rows(table, idx):
    return _make_combine_gather_kernel()(table, idx)


_NJ = 1


def _ffn_body(xin_ref, w1_ref, b1_ref, w2_ref, b2_ref, out_ref):
    xin = _unpack_bf16(xin_ref[...])
    h = jnp.maximum(
        jnp.dot(xin, w1_ref[0], preferred_element_type=jnp.float32)
        + b1_ref[0], 0.0)
    part = jnp.dot(h, w2_ref[0], preferred_element_type=jnp.float32)
    out_ref[...] = _pack_bf16(part + b2_ref[0])


def _ffn(buf, w1, b1r, w2, b2r, interpret=False):
    return pl.pallas_call(
        _ffn_body,
        grid=(_E,),
        in_specs=[
            pl.BlockSpec((_C, _DP), lambda e: (e, 0)),
            pl.BlockSpec((1, _D, _F), lambda e: (e, 0, 0)),
            pl.BlockSpec((1, 1, _F), lambda e: (e, 0, 0)),
            pl.BlockSpec((1, _F, _D), lambda e: (e, 0, 0)),
            pl.BlockSpec((1, 1, _D), lambda e: (e, 0, 0)),
        ],
        out_specs=pl.BlockSpec((_C, _DP), lambda e: (e, 0)),
        out_shape=jax.ShapeDtypeStruct((_EC, _DP), jnp.float32),
        interpret=interpret,
    )(buf, w1, b1r, w2, b2r)


_TBLK = 512


def _combine_body(g0_ref, g1_ref, wc_ref, y_ref):
    w0 = wc_ref[:, 0:1]
    w1 = wc_ref[:, 1:2]
    g0 = _unpack_bf16(g0_ref[...])
    g1 = _unpack_bf16(g1_ref[...])
    y_ref[...] = (jnp.where(w0 > 0, w0 * g0, 0.0)
                  + jnp.where(w1 > 0, w1 * g1, 0.0))


def _combine(g, wc, interpret=False):
    nb = _T // _TBLK
    return pl.pallas_call(
        _combine_body,
        grid=(nb,),
        in_specs=[
            pl.BlockSpec((_TBLK, _DP), lambda t: (t, 0)),
            pl.BlockSpec((_TBLK, _DP), lambda t: (t + nb, 0)),
            pl.BlockSpec((_TBLK, _K), lambda t: (t, 0)),
        ],
        out_specs=pl.BlockSpec((_TBLK, _D), lambda t: (t, 0)),
        out_shape=jax.ShapeDtypeStruct((_T, _D), jnp.float32),
        interpret=interpret,
    )(g, g, wc)


def kernel(x, wg, w1, b1, w2, b2):
    fi, kf, wc, xb = _gate(x, wg)
    buf = _dispatch(xb, fi, kf)
    eo = _ffn(buf, w1, b1.reshape(_E, 1, _F), w2, b2.reshape(_E, 1, _D))
    g = _gather_rows(eo, fi)
    return _combine(g, wc)


# speedup vs baseline: 1.0349x; 1.0005x over previous
import dataclasses
import functools

import jax
import jax.numpy as jnp
from jax.experimental import pallas as pl
from jax.experimental.pallas import tpu as pltpu
from jax.experimental.pallas import tpu_sc as plsc

_T = 2048
_D = 1024
_E = 8
_F = 2048
_K = 2
_C = (_T * _K) // _E
_EC = _E * _C

_NUM_WORKERS = 32
_ROWS_PER_CHUNK = 64
_DP = _D // 2


def _pack_bf16(a):
    n2 = a.shape[1] // 2
    b = a.astype(jnp.bfloat16)
    lo = (jax.lax.bitcast_convert_type(b[:, :n2], jnp.int16)
          .astype(jnp.int32) & 0xFFFF)
    hi = jax.lax.bitcast_convert_type(b[:, n2:], jnp.int16).astype(jnp.int32)
    return jax.lax.bitcast_convert_type(lo | (hi << 16), jnp.float32)


def _unpack_bf16(p):
    wi = jax.lax.bitcast_convert_type(p, jnp.int32)
    lo = jax.lax.bitcast_convert_type(
        (wi & 0xFFFF).astype(jnp.int16), jnp.bfloat16).astype(jnp.float32)
    hi = jax.lax.bitcast_convert_type(
        jax.lax.shift_right_logical(wi, 16).astype(jnp.int16),
        jnp.bfloat16).astype(jnp.float32)
    return jnp.concatenate([lo, hi], axis=1)


def _cumsum_lanes(a):
    n = a.shape[1]
    k = 1
    while k < n:
        shifted = jnp.concatenate(
            [jnp.zeros((a.shape[0], k), a.dtype), a[:, :-k]], axis=1)
        a = a + shifted
        k *= 2
    return a


def _gate_body(x_ref, wg_ref, fi_ref, kf_ref, wc_ref, xb_ref):
    x = x_ref[...]
    xb_ref[...] = _pack_bf16(x)
    wg = wg_ref[...]
    logits = jnp.dot(x, wg, preferred_element_type=jnp.float32)
    lt = logits.T
    mx = jnp.max(lt, axis=0, keepdims=True)
    eg = jnp.exp(lt - mx)
    gates = eg / jnp.sum(eg, axis=0, keepdims=True)

    iota = jax.lax.broadcasted_iota(jnp.int32, (_E, _T), 0)
    m1 = jnp.max(gates, axis=0, keepdims=True)
    idx1 = jnp.min(jnp.where(gates == m1, iota, _E), axis=0, keepdims=True)
    g2 = jnp.where(iota == idx1, -jnp.inf, gates)
    m2 = jnp.max(g2, axis=0, keepdims=True)
    idx2 = jnp.min(jnp.where(g2 == m2, iota, _E), axis=0, keepdims=True)

    s = m1 + m2 + 1e-9
    w1v = m1 / s
    w2v = m2 / s

    oh1 = (iota == idx1).astype(jnp.int32)
    oh2 = (iota == idx2).astype(jnp.int32)
    inc1 = _cumsum_lanes(oh1)
    inc2 = _cumsum_lanes(oh2)
    cnt1 = inc1[:, _T - 1:_T]
    pos1 = jnp.sum(jnp.where(oh1 == 1, inc1 - 1, 0), axis=0, keepdims=True)
    pos2 = jnp.sum(jnp.where(oh2 == 1, inc2 - 1 + cnt1, 0), axis=0,
                   keepdims=True)

    keep1 = pos1 < _C
    keep2 = pos2 < _C
    fi1 = jnp.where(keep1, idx1 * _C + pos1, _EC - 1)
    fi2 = jnp.where(keep2, idx2 * _C + pos2, _EC - 1)

    fi_ref[pl.ds(0, _T)] = fi1.reshape(_T)
    fi_ref[pl.ds(_T, _T)] = fi2.reshape(_T)
    kf_ref[pl.ds(0, _T)] = keep1.astype(jnp.int32).reshape(_T)
    kf_ref[pl.ds(_T, _T)] = keep2.astype(jnp.int32).reshape(_T)
    wcT = jnp.concatenate([jnp.where(keep1, w1v, 0.0),
                           jnp.where(keep2, w2v, 0.0)], axis=0)
    wc_ref[...] = wcT.T


def _gate(x, wg, interpret=False):
    return pl.pallas_call(
        _gate_body,
        out_shape=(jax.ShapeDtypeStruct((_K * _T,), jnp.int32),
                   jax.ShapeDtypeStruct((_K * _T,), jnp.int32),
                   jax.ShapeDtypeStruct((_T, _K), jnp.float32),
                   jax.ShapeDtypeStruct((_T, _DP), jnp.float32)),
        interpret=interpret,
    )(x, wg)


def _sc_compiler_params():
    cp = pltpu.CompilerParams()
    if "needs_layout_passes" in pltpu.CompilerParams.__dataclass_fields__:
        cp = dataclasses.replace(cp, needs_layout_passes=False)
    return cp


def _mesh():
    return plsc.VectorSubcoreMesh(core_axis_name="c", subcore_axis_name="s")


def _make_dispatch_kernel():
    per_worker = _EC // _NUM_WORKERS
    ch = _ROWS_PER_CHUNK
    n_chunks = per_worker // ch

    @functools.partial(
        pl.kernel,
        out_type=jax.ShapeDtypeStruct((_EC, _DP), jnp.float32),
        mesh=_mesh(),
        compiler_params=_sc_compiler_params(),
        scratch_types=[pltpu.VMEM((_EC,), jnp.int32),
                       pltpu.VMEM((_EC,), jnp.int32),
                       pltpu.VMEM((_EC,), jnp.int32),
                       pltpu.VMEM((ch, _DP), jnp.float32),
                       pltpu.VMEM((ch, _DP), jnp.float32),
                       pltpu.SemaphoreType.DMA,
                       pltpu.SemaphoreType.DMA,
                       pltpu.SemaphoreType.DMA,
                       pltpu.SemaphoreType.DMA],
    )
    def dispatch_kernel(x_hbm, fi_hbm, kf_hbm, buf_hbm,
                        fi_v, kf_v, inv_v, b0, b1, gs0, gs1, ws0, ws1):
        wid = jax.lax.axis_index("s") * 2 + jax.lax.axis_index("c")
        base = wid * per_worker
        pltpu.sync_copy(fi_hbm, fi_v)
        pltpu.sync_copy(kf_hbm, kf_v)

        @pl.loop(0, per_worker // 16)
        def _(i):
            inv_v[pl.ds(base + i * 16, 16)] = jnp.zeros((16,), jnp.int32)

        for s in (0, 1):
            @pl.loop(0, _T // 16)
            def _(i, s=s):
                eb = i * 16
                f = fi_v[pl.ds(s * _T + eb, 16)]
                kf = kf_v[pl.ds(s * _T + eb, 16)]
                tok = jax.lax.iota(jnp.int32, 16) + eb
                plsc.store_scatter(inv_v, [f], tok, mask=kf == 1)

        my_idx = inv_v.at[pl.ds(base, per_worker)]
        bufs = (b0, b1)
        gsems = (gs0, gs1)
        wsems = (ws0, ws1)
        writes = [None, None]
        gathers = [None, None]
        for c in range(n_chunks):
            s = c % 2
            if writes[s] is not None:
                writes[s].wait()
            gathers[s] = pltpu.async_copy(
                x_hbm.at[my_idx.at[pl.ds(c * ch, ch)]], bufs[s], gsems[s])
            if c % 2 == 1:
                for s2 in (0, 1):
                    gathers[s2].wait()
                    writes[s2] = pltpu.async_copy(
                        bufs[s2],
                        buf_hbm.at[pl.ds(base + (c - 1 + s2) * ch, ch)],
                        wsems[s2])
        for s2 in (0, 1):
            if writes[s2] is not None:
                writes[s2].wait()

    return dispatch_kernel


def _dispatch(x, fi_flat, kf_flat):
    return _make_dispatch_kernel()(x, fi_flat, kf_flat)


def _make_combine_gather_kernel():
    per_worker = _EC // _NUM_WORKERS
    ch = _ROWS_PER_CHUNK
    n_chunks = per_worker // ch

    @functools.partial(
        pl.kernel,
        out_type=jax.ShapeDtypeStruct((_EC, _DP), jnp.float32),
        mesh=_mesh(),
        compiler_params=_sc_compiler_params(),
        scratch_types=[pltpu.VMEM((per_worker,), jnp.int32),
                       pltpu.VMEM((ch, _DP), jnp.float32),
                       pltpu.VMEM((ch, _DP), jnp.float32),
                       pltpu.SemaphoreType.DMA,
                       pltpu.SemaphoreType.DMA,
                       pltpu.SemaphoreType.DMA,
                       pltpu.SemaphoreType.DMA],
    )
    def combine_kernel(eo_hbm, fi_hbm, out_hbm,
                       idx_v, b0, b1, gs0, gs1, ws0, ws1):
        wid = jax.lax.axis_index("s") * 2 + jax.lax.axis_index("c")
        base = wid * per_worker

        pltpu.sync_copy(fi_hbm.at[pl.ds(base, per_worker)], idx_v)
        bufs = (b0, b1)
        gsems = (gs0, gs1)
        wsems = (ws0, ws1)
        writes = [None, None]
        gathers = [None, None]
        for c in range(n_chunks):
            s = c % 2
            if writes[s] is not None:
                writes[s].wait()
            gathers[s] = pltpu.async_copy(
                eo_hbm.at[idx_v.at[pl.ds(c * ch, ch)]], bufs[s], gsems[s])
            if c % 2 == 1:
                for s2 in (0, 1):
                    gathers[s2].wait()
                    writes[s2] = pltpu.async_copy(
                        bufs[s2],
                        out_hbm.at[pl.ds(base + (c - 1 + s2) * ch, ch)],
                        wsems[s2])
        for s2 in (0, 1):
            if writes[s2] is not None:
                writes[s2].wait()

    return combine_kernel


def _gather_rows(table, idx):
    return _make_combine_gather_kernel()(table, idx)


_NJ = 1


def _ffn_body(xin_ref, w1_ref, b1_ref, w2_ref, b2_ref, out_ref):
    xin = _unpack_bf16(xin_ref[...])
    h = jnp.maximum(
        jnp.dot(xin, w1_ref[0], preferred_element_type=jnp.float32)
        + b1_ref[0], 0.0)
    part = jnp.dot(h, w2_ref[0], preferred_element_type=jnp.float32)
    out_ref[...] = _pack_bf16(part + b2_ref[0])


def _ffn(buf, w1, b1r, w2, b2r, interpret=False):
    return pl.pallas_call(
        _ffn_body,
        grid=(_E,),
        in_specs=[
            pl.BlockSpec((_C, _DP), lambda e: (e, 0)),
            pl.BlockSpec((1, _D, _F), lambda e: (e, 0, 0)),
            pl.BlockSpec((1, 1, _F), lambda e: (e, 0, 0)),
            pl.BlockSpec((1, _F, _D), lambda e: (e, 0, 0)),
            pl.BlockSpec((1, 1, _D), lambda e: (e, 0, 0)),
        ],
        out_specs=pl.BlockSpec((_C, _DP), lambda e: (e, 0)),
        out_shape=jax.ShapeDtypeStruct((_EC, _DP), jnp.float32),
        interpret=interpret,
    )(buf, w1, b1r, w2, b2r)


_TBLK = 512


def _combine_body(g0_ref, g1_ref, wc_ref, y_ref):
    w0 = wc_ref[:, 0:1]
    w1 = wc_ref[:, 1:2]
    g0 = _unpack_bf16(g0_ref[...])
    g1 = _unpack_bf16(g1_ref[...])
    y_ref[...] = (jnp.where(w0 > 0, w0 * g0, 0.0)
                  + jnp.where(w1 > 0, w1 * g1, 0.0))


def _combine(g, wc, interpret=False):
    nb = _T // _TBLK
    return pl.pallas_call(
        _combine_body,
        grid=(nb,),
        in_specs=[
            pl.BlockSpec((_TBLK, _DP), lambda t: (t, 0)),
            pl.BlockSpec((_TBLK, _DP), lambda t: (t + nb, 0)),
            pl.BlockSpec((_TBLK, _K), lambda t: (t, 0)),
        ],
        out_specs=pl.BlockSpec((_TBLK, _D), lambda t: (t, 0)),
        out_shape=jax.ShapeDtypeStruct((_T, _D), jnp.float32),
        interpret=interpret,
    )(g, g, wc)


def kernel(x, wg, w1, b1, w2, b2):
    fi, kf, wc, xb = _gate(x, wg)
    buf = _dispatch(xb, fi, kf)
    eo = _ffn(buf, w1, b1.reshape(_E, 1, _F), w2, b2.reshape(_E, 1, _D))
    g = _gather_rows(eo, fi)
    return _combine(g, wc)


# speedup vs baseline: 1.0415x; 1.0064x over previous
import dataclasses
import functools

import jax
import jax.numpy as jnp
from jax.experimental import pallas as pl
from jax.experimental.pallas import tpu as pltpu
from jax.experimental.pallas import tpu_sc as plsc

_T = 2048
_D = 1024
_E = 8
_F = 2048
_K = 2
_C = (_T * _K) // _E
_EC = _E * _C

_NUM_WORKERS = 32
_ROWS_PER_CHUNK = 64
_DP = _D // 2


def _pack_bf16(a):
    n2 = a.shape[1] // 2
    b = a.astype(jnp.bfloat16)
    lo = (jax.lax.bitcast_convert_type(b[:, :n2], jnp.int16)
          .astype(jnp.int32) & 0xFFFF)
    hi = jax.lax.bitcast_convert_type(b[:, n2:], jnp.int16).astype(jnp.int32)
    return jax.lax.bitcast_convert_type(lo | (hi << 16), jnp.float32)


def _unpack_bf16(p):
    wi = jax.lax.bitcast_convert_type(p, jnp.int32)
    lo = jax.lax.bitcast_convert_type(
        (wi & 0xFFFF).astype(jnp.int16), jnp.bfloat16).astype(jnp.float32)
    hi = jax.lax.bitcast_convert_type(
        jax.lax.shift_right_logical(wi, 16).astype(jnp.int16),
        jnp.bfloat16).astype(jnp.float32)
    return jnp.concatenate([lo, hi], axis=1)


def _cumsum_lanes(a):
    n = a.shape[1]
    k = 1
    while k < n:
        shifted = jnp.concatenate(
            [jnp.zeros((a.shape[0], k), a.dtype), a[:, :-k]], axis=1)
        a = a + shifted
        k *= 2
    return a


def _gate_body(x_ref, wg_ref, fi_ref, kf_ref, wc_ref, xb_ref):
    x = x_ref[...]
    xb_ref[...] = _pack_bf16(x)
    wg = wg_ref[...]
    logits = jnp.dot(x, wg, preferred_element_type=jnp.float32)
    lt = logits.T
    mx = jnp.max(lt, axis=0, keepdims=True)
    eg = jnp.exp(lt - mx)
    gates = eg / jnp.sum(eg, axis=0, keepdims=True)

    iota = jax.lax.broadcasted_iota(jnp.int32, (_E, _T), 0)
    m1 = jnp.max(gates, axis=0, keepdims=True)
    idx1 = jnp.min(jnp.where(gates == m1, iota, _E), axis=0, keepdims=True)
    g2 = jnp.where(iota == idx1, -jnp.inf, gates)
    m2 = jnp.max(g2, axis=0, keepdims=True)
    idx2 = jnp.min(jnp.where(g2 == m2, iota, _E), axis=0, keepdims=True)

    s = m1 + m2 + 1e-9
    w1v = m1 / s
    w2v = m2 / s

    oh1 = (iota == idx1).astype(jnp.int32)
    oh2 = (iota == idx2).astype(jnp.int32)
    inc1 = _cumsum_lanes(oh1)
    inc2 = _cumsum_lanes(oh2)
    cnt1 = inc1[:, _T - 1:_T]
    pos1 = jnp.sum(jnp.where(oh1 == 1, inc1 - 1, 0), axis=0, keepdims=True)
    pos2 = jnp.sum(jnp.where(oh2 == 1, inc2 - 1 + cnt1, 0), axis=0,
                   keepdims=True)

    keep1 = pos1 < _C
    keep2 = pos2 < _C
    slot1 = idx1 * _C + pos1
    slot2 = idx2 * _C + pos2
    fi1 = jnp.where(keep1, slot1, _EC - 1)
    fi2 = jnp.where(keep2, slot2, _EC - 1)
    fs1 = jnp.where(keep1, slot1, _EC)
    fs2 = jnp.where(keep2, slot2, _EC)

    fi_ref[pl.ds(0, _T)] = fi1.reshape(_T)
    fi_ref[pl.ds(_T, _T)] = fi2.reshape(_T)
    kf_ref[pl.ds(0, _T)] = fs1.reshape(_T)
    kf_ref[pl.ds(_T, _T)] = fs2.reshape(_T)
    wcT = jnp.concatenate([jnp.where(keep1, w1v, 0.0),
                           jnp.where(keep2, w2v, 0.0)], axis=0)
    wc_ref[...] = wcT.T


def _gate(x, wg, interpret=False):
    return pl.pallas_call(
        _gate_body,
        out_shape=(jax.ShapeDtypeStruct((_K * _T,), jnp.int32),
                   jax.ShapeDtypeStruct((_K * _T,), jnp.int32),
                   jax.ShapeDtypeStruct((_T, _K), jnp.float32),
                   jax.ShapeDtypeStruct((_T, _DP), jnp.float32)),
        interpret=interpret,
    )(x, wg)


def _sc_compiler_params():
    cp = pltpu.CompilerParams()
    if "needs_layout_passes" in pltpu.CompilerParams.__dataclass_fields__:
        cp = dataclasses.replace(cp, needs_layout_passes=False)
    return cp


def _mesh():
    return plsc.VectorSubcoreMesh(core_axis_name="c", subcore_axis_name="s")


def _make_dispatch_kernel():
    per_worker = _EC // _NUM_WORKERS
    ch = _ROWS_PER_CHUNK
    n_chunks = per_worker // ch

    @functools.partial(
        pl.kernel,
        out_type=jax.ShapeDtypeStruct((_EC, _DP), jnp.float32),
        mesh=_mesh(),
        compiler_params=_sc_compiler_params(),
        scratch_types=[pltpu.VMEM((_EC,), jnp.int32),
                       pltpu.VMEM((_EC,), jnp.int32),
                       pltpu.VMEM((ch, _DP), jnp.float32),
                       pltpu.VMEM((ch, _DP), jnp.float32),
                       pltpu.SemaphoreType.DMA,
                       pltpu.SemaphoreType.DMA,
                       pltpu.SemaphoreType.DMA,
                       pltpu.SemaphoreType.DMA],
    )
    def dispatch_kernel(x_hbm, fs_hbm, buf_hbm,
                        fs_v, inv_v, b0, b1, gs0, gs1, ws0, ws1):
        wid = jax.lax.axis_index("s") * 2 + jax.lax.axis_index("c")
        base = wid * per_worker
        pltpu.sync_copy(fs_hbm, fs_v)

        @pl.loop(0, per_worker // 16)
        def _(i):
            inv_v[pl.ds(base + i * 16, 16)] = jnp.zeros((16,), jnp.int32)

        for s in (0, 1):
            @pl.loop(0, _T // 32)
            def _(i, s=s):
                for u in (0, 1):
                    eb = i * 32 + u * 16
                    f = fs_v[pl.ds(s * _T + eb, 16)]
                    tok = jax.lax.iota(jnp.int32, 16) + eb
                    plsc.store_scatter(inv_v, [f], tok, mask=f < _EC)

        my_idx = inv_v.at[pl.ds(base, per_worker)]
        bufs = (b0, b1)
        gsems = (gs0, gs1)
        wsems = (ws0, ws1)
        writes = [None, None]
        gathers = [None, None]
        for c in range(n_chunks):
            s = c % 2
            if writes[s] is not None:
                writes[s].wait()
            gathers[s] = pltpu.async_copy(
                x_hbm.at[my_idx.at[pl.ds(c * ch, ch)]], bufs[s], gsems[s])
            if c % 2 == 1:
                for s2 in (0, 1):
                    gathers[s2].wait()
                    writes[s2] = pltpu.async_copy(
                        bufs[s2],
                        buf_hbm.at[pl.ds(base + (c - 1 + s2) * ch, ch)],
                        wsems[s2])
        for s2 in (0, 1):
            if writes[s2] is not None:
                writes[s2].wait()

    return dispatch_kernel


def _dispatch(x, fs_flat):
    return _make_dispatch_kernel()(x, fs_flat)


def _make_combine_gather_kernel():
    per_worker = _EC // _NUM_WORKERS
    ch = _ROWS_PER_CHUNK
    n_chunks = per_worker // ch

    @functools.partial(
        pl.kernel,
        out_type=jax.ShapeDtypeStruct((_EC, _DP), jnp.float32),
        mesh=_mesh(),
        compiler_params=_sc_compiler_params(),
        scratch_types=[pltpu.VMEM((per_worker,), jnp.int32),
                       pltpu.VMEM((ch, _DP), jnp.float32),
                       pltpu.VMEM((ch, _DP), jnp.float32),
                       pltpu.SemaphoreType.DMA,
                       pltpu.SemaphoreType.DMA,
                       pltpu.SemaphoreType.DMA,
                       pltpu.SemaphoreType.DMA],
    )
    def combine_kernel(eo_hbm, fi_hbm, out_hbm,
                       idx_v, b0, b1, gs0, gs1, ws0, ws1):
        wid = jax.lax.axis_index("s") * 2 + jax.lax.axis_index("c")
        base = wid * per_worker

        pltpu.sync_copy(fi_hbm.at[pl.ds(base, per_worker)], idx_v)
        bufs = (b0, b1)
        gsems = (gs0, gs1)
        wsems = (ws0, ws1)
        writes = [None, None]
        gathers = [None, None]
        for c in range(n_chunks):
            s = c % 2
            if writes[s] is not None:
                writes[s].wait()
            gathers[s] = pltpu.async_copy(
                eo_hbm.at[idx_v.at[pl.ds(c * ch, ch)]], bufs[s], gsems[s])
            if c % 2 == 1:
                for s2 in (0, 1):
                    gathers[s2].wait()
                    writes[s2] = pltpu.async_copy(
                        bufs[s2],
                        out_hbm.at[pl.ds(base + (c - 1 + s2) * ch, ch)],
                        wsems[s2])
        for s2 in (0, 1):
            if writes[s2] is not None:
                writes[s2].wait()

    return combine_kernel


def _gather_rows(table, idx):
    return _make_combine_gather_kernel()(table, idx)


_NJ = 1


def _ffn_body(xin_ref, w1_ref, b1_ref, w2_ref, b2_ref, out_ref):
    xin = _unpack_bf16(xin_ref[...])
    h = jnp.maximum(
        jnp.dot(xin, w1_ref[0], preferred_element_type=jnp.float32)
        + b1_ref[0], 0.0)
    part = jnp.dot(h, w2_ref[0], preferred_element_type=jnp.float32)
    out_ref[...] = _pack_bf16(part + b2_ref[0])


def _ffn(buf, w1, b1r, w2, b2r, interpret=False):
    return pl.pallas_call(
        _ffn_body,
        grid=(_E,),
        in_specs=[
            pl.BlockSpec((_C, _DP), lambda e: (e, 0)),
            pl.BlockSpec((1, _D, _F), lambda e: (e, 0, 0)),
            pl.BlockSpec((1, 1, _F), lambda e: (e, 0, 0)),
            pl.BlockSpec((1, _F, _D), lambda e: (e, 0, 0)),
            pl.BlockSpec((1, 1, _D), lambda e: (e, 0, 0)),
        ],
        out_specs=pl.BlockSpec((_C, _DP), lambda e: (e, 0)),
        out_shape=jax.ShapeDtypeStruct((_EC, _DP), jnp.float32),
        interpret=interpret,
    )(buf, w1, b1r, w2, b2r)


_TBLK = 512


def _combine_body(g0_ref, g1_ref, wc_ref, y_ref):
    w0 = wc_ref[:, 0:1]
    w1 = wc_ref[:, 1:2]
    g0 = _unpack_bf16(g0_ref[...])
    g1 = _unpack_bf16(g1_ref[...])
    y_ref[...] = (jnp.where(w0 > 0, w0 * g0, 0.0)
                  + jnp.where(w1 > 0, w1 * g1, 0.0))


def _combine(g, wc, interpret=False):
    nb = _T // _TBLK
    return pl.pallas_call(
        _combine_body,
        grid=(nb,),
        in_specs=[
            pl.BlockSpec((_TBLK, _DP), lambda t: (t, 0)),
            pl.BlockSpec((_TBLK, _DP), lambda t: (t + nb, 0)),
            pl.BlockSpec((_TBLK, _K), lambda t: (t, 0)),
        ],
        out_specs=pl.BlockSpec((_TBLK, _D), lambda t: (t, 0)),
        out_shape=jax.ShapeDtypeStruct((_T, _D), jnp.float32),
        interpret=interpret,
    )(g, g, wc)


def kernel(x, wg, w1, b1, w2, b2):
    fi, fs, wc, xb = _gate(x, wg)
    buf = _dispatch(xb, fs)
    eo = _ffn(buf, w1, b1.reshape(_E, 1, _F), w2, b2.reshape(_E, 1, _D))
    g = _gather_rows(eo, fi)
    return _combine(g, wc)
